# Initial kernel scaffold; baseline (speedup 1.0000x reference)
import functools

import jax
import jax.numpy as jnp
from jax import lax
from jax.experimental import pallas as pl
from jax.experimental.pallas import tpu as pltpu
from jax.experimental.pallas import tpu_sc as plsc

N = 10000
D = 128
E = 320000
G = 64
FFN = 300
NC, NS, LANES = 2, 16, 16
NW = NC * NS
CH = 128
CPW = -(-E // (NW * CH))
EPW = CPW * CH
EP = EPW * NW
NP = 10240
RPS = NP // NS
TRASH = N
GP = 128
DEGW = 16
ZROWS = 128
RPW = NP // NW
PCH = 64
PCPW = RPW // PCH

_mesh = plsc.VectorSubcoreMesh(
    core_axis_name="c", subcore_axis_name="s", num_cores=NC, num_subcores=NS)

_f32 = jnp.float32


def _fill(buf, rows, width, value):
    vec = jnp.full((LANES,), value, _f32)

    def body(i, _):
        for k in range(width // LANES):
            buf[i, pl.ds(k * LANES, LANES)] = vec
        return 0

    lax.fori_loop(0, rows, body, 0)


@functools.partial(
    pl.kernel,
    out_type=jax.ShapeDtypeStruct((NC, NP, DEGW), _f32),
    mesh=_mesh,
    scratch_types=[
        pltpu.VMEM_SHARED((NP, DEGW), _f32),
        pltpu.VMEM((1, CH), jnp.int32),
        pltpu.VMEM((CH, DEGW), _f32),
        pltpu.VMEM((ZROWS, DEGW), _f32),
    ],
)
def _deg_kernel(dst_hbm, out_hbm, acc_sh, idx_v, ones_v, zbuf):
    cid = lax.axis_index("c")
    sid = lax.axis_index("s")
    w = cid * NS + sid
    _fill(ones_v, CH, DEGW, 1.0)
    _fill(zbuf, ZROWS, DEGW, 0.0)
    r0 = sid * RPS
    for t in range(RPS // ZROWS):
        pltpu.sync_copy(zbuf, acc_sh.at[pl.ds(r0 + t * ZROWS, ZROWS)])
    plsc.subcore_barrier()

    def step(j, _):
        base = w * EPW + j * CH
        pltpu.sync_copy(dst_hbm.at[pl.ds(base, CH)], idx_v.at[0])
        pltpu.sync_copy(ones_v, acc_sh.at[idx_v.at[0]], add=True)
        return 0

    lax.fori_loop(0, CPW, step, 0)
    plsc.subcore_barrier()
    pltpu.sync_copy(acc_sh.at[pl.ds(r0, RPS)],
                    out_hbm.at[cid, pl.ds(r0, RPS)])


@functools.partial(
    pl.kernel,
    out_type=jax.ShapeDtypeStruct((NC, NP, D), _f32),
    mesh=_mesh,
    scratch_types=[
        pltpu.VMEM_SHARED((NP, D), _f32),
        pltpu.VMEM((ZROWS, D), _f32),
        pltpu.VMEM((2, CH), jnp.int32),
        pltpu.VMEM((2, CH), jnp.int32),
        pltpu.VMEM((2, CH, D), _f32),
        pltpu.SemaphoreType.DMA,
        pltpu.SemaphoreType.DMA,
    ],
)
def _edge_kernel(ms_hbm, src_hbm, dst_hbm, out_hbm,
                 acc_sh, zbuf, idx_s, idx_d, rows_v, gsem0, gsem1):
    cid = lax.axis_index("c")
    sid = lax.axis_index("s")
    w = cid * NS + sid
    r0 = sid * RPS

    @pl.when(cid == 0)
    def _():
        pltpu.sync_copy(ms_hbm.at[pl.ds(r0, RPS)], acc_sh.at[pl.ds(r0, RPS)])

    @pl.when(cid != 0)
    def _():
        _fill(zbuf, ZROWS, D, 0.0)
        for t in range(RPS // ZROWS):
            pltpu.sync_copy(zbuf, acc_sh.at[pl.ds(r0 + t * ZROWS, ZROWS)])

    plsc.subcore_barrier()

    gsems = (gsem0, gsem1)

    def start(j, slot):
        base = w * EPW + j * CH
        pltpu.sync_copy(src_hbm.at[pl.ds(base, CH)], idx_s.at[slot])
        pltpu.sync_copy(dst_hbm.at[pl.ds(base, CH)], idx_d.at[slot])
        return pltpu.async_copy(ms_hbm.at[idx_s.at[slot]], rows_v.at[slot],
                                gsems[slot])

    cp = start(0, 0)
    for j in range(CPW):
        nxt = start(j + 1, (j + 1) % 2) if j + 1 < CPW else None
        cp.wait()
        slot = j % 2
        pltpu.sync_copy(rows_v.at[slot], acc_sh.at[idx_d.at[slot]], add=True)
        cp = nxt

    plsc.subcore_barrier()
    pltpu.sync_copy(acc_sh.at[pl.ds(r0, RPS)],
                    out_hbm.at[cid, pl.ds(r0, RPS)])


@functools.partial(
    pl.kernel,
    out_type=(jax.ShapeDtypeStruct((NC, GP, D), _f32),
              jax.ShapeDtypeStruct((NC, GP, DEGW), _f32)),
    mesh=_mesh,
    scratch_types=[
        pltpu.VMEM_SHARED((GP, D), _f32),
        pltpu.VMEM_SHARED((GP, DEGW), _f32),
        pltpu.VMEM((1, PCH), jnp.int32),
        pltpu.VMEM((PCH, D), _f32),
        pltpu.VMEM((PCH, DEGW), _f32),
        pltpu.VMEM((GP // NS, D), _f32),
        pltpu.VMEM((GP // NS, DEGW), _f32),
    ],
)
def _pool_kernel(h_hbm, batch_hbm, sums_hbm, cnt_hbm,
                 sums_sh, cnt_sh, idx_v, rows_v, ones_v, zs, zc):
    cid = lax.axis_index("c")
    sid = lax.axis_index("s")
    w = cid * NS + sid
    gps = GP // NS
    _fill(ones_v, PCH, DEGW, 1.0)
    _fill(zs, gps, D, 0.0)
    _fill(zc, gps, DEGW, 0.0)
    g0 = sid * gps
    pltpu.sync_copy(zs, sums_sh.at[pl.ds(g0, gps)])
    pltpu.sync_copy(zc, cnt_sh.at[pl.ds(g0, gps)])
    plsc.subcore_barrier()

    def step(j, _):
        base = w * RPW + j * PCH
        pltpu.sync_copy(batch_hbm.at[pl.ds(base, PCH)], idx_v.at[0])
        pltpu.sync_copy(h_hbm.at[pl.ds(base, PCH)], rows_v)
        pltpu.sync_copy(rows_v, sums_sh.at[idx_v.at[0]], add=True)
        pltpu.sync_copy(ones_v, cnt_sh.at[idx_v.at[0]], add=True)
        return 0

    lax.fori_loop(0, PCPW, step, 0)
    plsc.subcore_barrier()
    pltpu.sync_copy(sums_sh.at[pl.ds(g0, gps)],
                    sums_hbm.at[cid, pl.ds(g0, gps)])
    pltpu.sync_copy(cnt_sh.at[pl.ds(g0, gps)],
                    cnt_hbm.at[cid, pl.ds(g0, gps)])


def _dot(a, b):
    return jnp.dot(a, b, preferred_element_type=_f32)


def _enc_body(x_ref, nw_ref, nb_ref, w0_ref, dp_ref, dinv_ref, ms0_ref):
    h0 = _dot(x_ref[...], nw_ref[...]) + nb_ref[...]
    deg = 1.0 + dp_ref[0, :, :1] + dp_ref[1, :, :1]
    rows = lax.broadcasted_iota(jnp.int32, (NP, 1), 0)
    dinv = lax.rsqrt(deg) * jnp.where(rows < N, 1.0, 0.0)
    dinv_ref[...] = dinv
    ms0_ref[...] = dinv * _dot(h0, w0_ref[...])


_enc = pl.pallas_call(
    _enc_body,
    out_shape=(jax.ShapeDtypeStruct((NP, 1), _f32),
               jax.ShapeDtypeStruct((NP, D), _f32)),
)


def _make_post(residual, has_next):
    def body(*refs):
        parts_ref, dinv_ref, b_ref, ga_ref, be_ref = refs[:5]
        i = 5
        hp_ref = wn_ref = None
        if residual:
            hp_ref = refs[i]; i += 1
        if has_next:
            wn_ref = refs[i]; i += 1
        h_ref = refs[i]; i += 1
        ms_ref = refs[i] if has_next else None

        s = parts_ref[0] + parts_ref[1]
        agg = dinv_ref[...] * s + b_ref[...]
        a = agg[:N]
        mean = jnp.mean(a, axis=0, keepdims=True)
        c = a - mean
        var = jnp.mean(c * c, axis=0, keepdims=True)
        hn = (agg - mean) * lax.rsqrt(var + 1e-5) * ga_ref[...] + be_ref[...]
        h = jnp.maximum(hn, 0.0)
        if residual:
            h = h + hp_ref[...]
        h_ref[...] = h
        if has_next:
            ms_ref[...] = dinv_ref[...] * _dot(h, wn_ref[...])

    out = [jax.ShapeDtypeStruct((NP, D), _f32)]
    if has_next:
        out.append(jax.ShapeDtypeStruct((NP, D), _f32))
    return pl.pallas_call(body, out_shape=tuple(out))


_post_first = _make_post(residual=False, has_next=True)
_post_mid = _make_post(residual=True, has_next=True)
_post_last = _make_post(residual=True, has_next=False)


def _mlp_body(sp_ref, cp_ref, w1_ref, b1_ref, w2_ref, b2_ref, w3_ref, b3_ref,
              out_ref):
    sums = sp_ref[0, :G] + sp_ref[1, :G]
    counts = cp_ref[0, :G, :1] + cp_ref[1, :G, :1]
    g = sums / jnp.maximum(counts, 1.0)
    g = jnp.maximum(_dot(g, w1_ref[...]) + b1_ref[...], 0.0)
    g = jnp.maximum(_dot(g, w2_ref[...]) + b2_ref[...], 0.0)
    out_ref[...] = _dot(g, w3_ref[...]) + b3_ref[...]


_mlp = pl.pallas_call(
    _mlp_body, out_shape=jax.ShapeDtypeStruct((G, 1), _f32))


def kernel(x, edge_index, edge_attr, batch, node_W, node_b, gcn_W, gcn_b,
           bn_gamma, bn_beta, mlp_W1, mlp_b1, mlp_W2, mlp_b2, mlp_W3, mlp_b3):
    del edge_attr
    x_pad = jnp.zeros((NP, D), _f32).at[:N].set(x)
    src = jnp.concatenate(
        [edge_index[0], jnp.zeros((EP - E,), jnp.int32)])
    dst = jnp.concatenate(
        [edge_index[1], jnp.full((EP - E,), TRASH, jnp.int32)])
    batch_pad = jnp.concatenate([batch, jnp.full((NP - N,), G, jnp.int32)])

    dparts = _deg_kernel(dst)
    dinv, ms = _enc(x_pad, node_W, node_b.reshape(1, D), gcn_W[0], dparts)

    h_prev = None
    for i in range(3):
        parts = _edge_kernel(ms, src, dst)
        b = gcn_b[i].reshape(1, D)
        ga = bn_gamma[i].reshape(1, D)
        be = bn_beta[i].reshape(1, D)
        if i == 0:
            h, ms = _post_first(parts, dinv, b, ga, be, gcn_W[1])
        elif i == 1:
            h, ms = _post_mid(parts, dinv, b, ga, be, h_prev, gcn_W[2])
        else:
            h = _post_last(parts, dinv, b, ga, be, h_prev)
        h_prev = h

    sums_p, cnt_p = _pool_kernel(h_prev, batch_pad)
    out = _mlp(sums_p, cnt_p, mlp_W1, mlp_b1.reshape(1, FFN),
               mlp_W2, mlp_b2.reshape(1, FFN), mlp_W3, mlp_b3.reshape(1, 1))
    return out


# speedup vs baseline: 9.3616x; 9.3616x over previous
import functools

import jax
import jax.numpy as jnp
from jax import lax
from jax.experimental import pallas as pl
from jax.experimental.pallas import tpu as pltpu
from jax.experimental.pallas import tpu_sc as plsc

N = 10000
D = 128
E = 320000
G = 64
FFN = 300
NC, NS, LANES = 2, 16, 16
NW = NC * NS
CH = 128
CPW = -(-E // (NW * CH))
EPW = CPW * CH
EP = EPW * NW
NP = 10240
RPS = NP // NS
TRASH = N
GP = 128
EZROWS = 64
RPW = NP // NW
PCH = 64
PCPW = RPW // PCH

_f32 = jnp.float32


@functools.lru_cache(maxsize=None)
def _mesh():
    return plsc.VectorSubcoreMesh(
        core_axis_name="c", subcore_axis_name="s",
        num_cores=NC, num_subcores=NS)


def _fill(buf, rows, width, value):
    vec = jnp.full((LANES,), value, _f32)

    def body(i, _):
        for k in range(width // LANES):
            buf[i, pl.ds(k * LANES, LANES)] = vec
        return 0

    lax.fori_loop(0, rows, body, 0)


@functools.lru_cache(maxsize=None)
def _get_deg_kernel():
    return functools.partial(
        pl.kernel,
        out_type=jax.ShapeDtypeStruct((NC, NP, D), _f32),
        mesh=_mesh(),
        scratch_types=[
            pltpu.VMEM_SHARED((NP, D), _f32),
            pltpu.VMEM((1, CH), jnp.int32),
            pltpu.VMEM((CH, D), _f32),
            pltpu.VMEM((EZROWS, D), _f32),
        ],
    )(_deg_body)


def _deg_body(dst_hbm, out_hbm, acc_sh, idx_v, ones_v, zbuf):
    cid = lax.axis_index("c")
    sid = lax.axis_index("s")
    w = cid * NS + sid
    _fill(ones_v, CH, D, 1.0)
    _fill(zbuf, EZROWS, D, 0.0)
    r0 = sid * RPS

    def zstep(t, _):
        pltpu.sync_copy(zbuf, acc_sh.at[pl.ds(r0 + t * EZROWS, EZROWS)])
        return 0

    lax.fori_loop(0, RPS // EZROWS, zstep, 0)
    plsc.subcore_barrier()

    def step(j, _):
        base = w * EPW + j * CH
        pltpu.sync_copy(dst_hbm.at[pl.ds(base, CH)], idx_v.at[0])
        pltpu.sync_copy(ones_v, acc_sh.at[idx_v.at[0]], add=True)
        return 0

    lax.fori_loop(0, CPW, step, 0)
    plsc.subcore_barrier()
    pltpu.sync_copy(acc_sh.at[pl.ds(r0, RPS)],
                    out_hbm.at[cid, pl.ds(r0, RPS)])


@functools.lru_cache(maxsize=None)
def _get_edge_kernel():
    return functools.partial(
        pl.kernel,
        out_type=jax.ShapeDtypeStruct((NC, NP, D), _f32),
        mesh=_mesh(),
        scratch_types=[
            pltpu.VMEM_SHARED((NP, D), _f32),
            pltpu.VMEM((EZROWS, D), _f32),
            pltpu.VMEM((2, CH), jnp.int32),
            pltpu.VMEM((2, CH), jnp.int32),
            pltpu.VMEM((2, CH, D), _f32),
            pltpu.SemaphoreType.DMA,
            pltpu.SemaphoreType.DMA,
        ],
    )(_edge_body)


def _edge_body(ms_hbm, src_hbm, dst_hbm, out_hbm,
               acc_sh, zbuf, idx_s, idx_d, rows_v, gsem0, gsem1):
    cid = lax.axis_index("c")
    sid = lax.axis_index("s")
    w = cid * NS + sid
    r0 = sid * RPS

    @pl.when(cid == 0)
    def _():
        pltpu.sync_copy(ms_hbm.at[pl.ds(r0, RPS)], acc_sh.at[pl.ds(r0, RPS)])

    @pl.when(cid != 0)
    def _():
        _fill(zbuf, EZROWS, D, 0.0)

        def zstep(t, _):
            pltpu.sync_copy(zbuf,
                            acc_sh.at[pl.ds(r0 + t * EZROWS, EZROWS)])
            return 0

        lax.fori_loop(0, RPS // EZROWS, zstep, 0)

    plsc.subcore_barrier()

    del gsem1

    def step(j, _):
        base = w * EPW + j * CH
        pltpu.sync_copy(src_hbm.at[pl.ds(base, CH)], idx_s.at[0])
        pltpu.sync_copy(dst_hbm.at[pl.ds(base, CH)], idx_d.at[0])
        pltpu.async_copy(ms_hbm.at[idx_s.at[0]], rows_v.at[0], gsem0).wait()
        pltpu.sync_copy(rows_v.at[0], acc_sh.at[idx_d.at[0]], add=True)
        return 0

    lax.fori_loop(0, CPW, step, 0)

    plsc.subcore_barrier()
    pltpu.sync_copy(acc_sh.at[pl.ds(r0, RPS)],
                    out_hbm.at[cid, pl.ds(r0, RPS)])


@functools.lru_cache(maxsize=None)
def _get_pool_kernel():
    return functools.partial(
        pl.kernel,
        out_type=(jax.ShapeDtypeStruct((NC, GP, D), _f32),
                  jax.ShapeDtypeStruct((NC, GP, D), _f32)),
        mesh=_mesh(),
        scratch_types=[
            pltpu.VMEM_SHARED((GP, D), _f32),
            pltpu.VMEM_SHARED((GP, D), _f32),
            pltpu.VMEM((1, PCH), jnp.int32),
            pltpu.VMEM((PCH, D), _f32),
            pltpu.VMEM((PCH, D), _f32),
            pltpu.VMEM((GP // NS, D), _f32),
        ],
    )(_pool_body)


def _pool_body(h_hbm, batch_hbm, sums_hbm, cnt_hbm,
               sums_sh, cnt_sh, idx_v, rows_v, ones_v, zs):
    cid = lax.axis_index("c")
    sid = lax.axis_index("s")
    w = cid * NS + sid
    gps = GP // NS
    _fill(ones_v, PCH, D, 1.0)
    _fill(zs, gps, D, 0.0)
    g0 = sid * gps
    pltpu.sync_copy(zs, sums_sh.at[pl.ds(g0, gps)])
    pltpu.sync_copy(zs, cnt_sh.at[pl.ds(g0, gps)])
    plsc.subcore_barrier()

    def step(j, _):
        base = w * RPW + j * PCH
        pltpu.sync_copy(batch_hbm.at[pl.ds(base, PCH)], idx_v.at[0])
        pltpu.sync_copy(h_hbm.at[pl.ds(base, PCH)], rows_v)
        pltpu.sync_copy(rows_v, sums_sh.at[idx_v.at[0]], add=True)
        pltpu.sync_copy(ones_v, cnt_sh.at[idx_v.at[0]], add=True)
        return 0

    lax.fori_loop(0, PCPW, step, 0)
    plsc.subcore_barrier()
    pltpu.sync_copy(sums_sh.at[pl.ds(g0, gps)],
                    sums_hbm.at[cid, pl.ds(g0, gps)])
    pltpu.sync_copy(cnt_sh.at[pl.ds(g0, gps)],
                    cnt_hbm.at[cid, pl.ds(g0, gps)])


def _dot(a, b):
    return jnp.dot(a, b, preferred_element_type=_f32)


def _enc_body(x_ref, nw_ref, nb_ref, w0_ref, dp_ref, dinv_ref, ms0_ref):
    h0 = _dot(x_ref[...], nw_ref[...]) + nb_ref[...]
    deg = 1.0 + dp_ref[0, :, :1] + dp_ref[1, :, :1]
    rows = lax.broadcasted_iota(jnp.int32, (NP, 1), 0)
    dinv = lax.rsqrt(deg) * jnp.where(rows < N, 1.0, 0.0)
    dinv_ref[...] = dinv
    ms0_ref[...] = dinv * _dot(h0, w0_ref[...])


_enc = pl.pallas_call(
    _enc_body,
    out_shape=(jax.ShapeDtypeStruct((NP, 1), _f32),
               jax.ShapeDtypeStruct((NP, D), _f32)),
)


def _make_post(residual, has_next):
    def body(*refs):
        parts_ref, dinv_ref, b_ref, ga_ref, be_ref = refs[:5]
        i = 5
        hp_ref = wn_ref = None
        if residual:
            hp_ref = refs[i]; i += 1
        if has_next:
            wn_ref = refs[i]; i += 1
        h_ref = refs[i]; i += 1
        ms_ref = refs[i] if has_next else None

        s = parts_ref[0] + parts_ref[1]
        agg = dinv_ref[...] * s + b_ref[...]
        a = agg[:N]
        mean = jnp.mean(a, axis=0, keepdims=True)
        c = a - mean
        var = jnp.mean(c * c, axis=0, keepdims=True)
        hn = (agg - mean) * lax.rsqrt(var + 1e-5) * ga_ref[...] + be_ref[...]
        h = jnp.maximum(hn, 0.0)
        if residual:
            h = h + hp_ref[...]
        h_ref[...] = h
        if has_next:
            ms_ref[...] = dinv_ref[...] * _dot(h, wn_ref[...])

    if has_next:
        out = (jax.ShapeDtypeStruct((NP, D), _f32),
               jax.ShapeDtypeStruct((NP, D), _f32))
    else:
        out = jax.ShapeDtypeStruct((NP, D), _f32)
    return pl.pallas_call(body, out_shape=out)


_post_first = _make_post(residual=False, has_next=True)
_post_mid = _make_post(residual=True, has_next=True)
_post_last = _make_post(residual=True, has_next=False)


def _mlp_body(sp_ref, cp_ref, w1_ref, b1_ref, w2_ref, b2_ref, w3_ref, b3_ref,
              out_ref):
    sums = sp_ref[0, :G] + sp_ref[1, :G]
    counts = cp_ref[0, :G, :1] + cp_ref[1, :G, :1]
    g = sums / jnp.maximum(counts, 1.0)
    g = jnp.maximum(_dot(g, w1_ref[...]) + b1_ref[...], 0.0)
    g = jnp.maximum(_dot(g, w2_ref[...]) + b2_ref[...], 0.0)
    out_ref[...] = _dot(g, w3_ref[...]) + b3_ref[...]


_mlp = pl.pallas_call(
    _mlp_body, out_shape=jax.ShapeDtypeStruct((G, 1), _f32))


def kernel(x, edge_index, edge_attr, batch, node_W, node_b, gcn_W, gcn_b,
           bn_gamma, bn_beta, mlp_W1, mlp_b1, mlp_W2, mlp_b2, mlp_W3, mlp_b3):
    del edge_attr
    x_pad = jnp.zeros((NP, D), _f32).at[:N].set(x)
    src = jnp.concatenate(
        [edge_index[0], jnp.zeros((EP - E,), jnp.int32)])
    dst = jnp.concatenate(
        [edge_index[1], jnp.full((EP - E,), TRASH, jnp.int32)])
    batch_pad = jnp.concatenate([batch, jnp.full((NP - N,), G, jnp.int32)])

    dparts = _get_deg_kernel()(dst)
    dinv, ms = _enc(x_pad, node_W, node_b.reshape(1, D), gcn_W[0], dparts)

    h_prev = None
    for i in range(3):
        parts = _get_edge_kernel()(ms, src, dst)
        b = gcn_b[i].reshape(1, D)
        ga = bn_gamma[i].reshape(1, D)
        be = bn_beta[i].reshape(1, D)
        if i == 0:
            h, ms = _post_first(parts, dinv, b, ga, be, gcn_W[1])
        elif i == 1:
            h, ms = _post_mid(parts, dinv, b, ga, be, h_prev, gcn_W[2])
        else:
            h = _post_last(parts, dinv, b, ga, be, h_prev)
        h_prev = h

    sums_p, cnt_p = _get_pool_kernel()(h_prev, batch_pad)
    out = _mlp(sums_p, cnt_p, mlp_W1, mlp_b1.reshape(1, FFN),
               mlp_W2, mlp_b2.reshape(1, FFN), mlp_W3, mlp_b3.reshape(1, 1))
    return out
